# Initial kernel scaffold; baseline (speedup 1.0000x reference)
import jax
import jax.numpy as jnp
from jax.experimental import pallas as pl


def kernel(x, edge_index, edge_type, W, W_self, b_type):
    raise NotImplementedError("write your pallas kernel here")


# speedup vs baseline: 8.6236x; 8.6236x over previous
import functools

import jax
import jax.numpy as jnp
from jax import lax
from jax.experimental import pallas as pl
from jax.experimental.pallas import tpu as pltpu
from jax.experimental.pallas import tpu_sc as plsc

_NC = 2
_NS = 16
_NW = _NC * _NS


def _htab_call(x, W, b_type, *, n_blk):
    n, d = x.shape
    t = b_type.shape[0]

    def body(x_ref, w_ref, b_ref, out_ref):
        h = lax.dot(
            x_ref[...],
            w_ref[...],
            precision=lax.Precision.HIGHEST,
            preferred_element_type=jnp.float32,
        )
        out_ref[...] = h[:, None, :] + b_ref[...][None, :, :]

    return pl.pallas_call(
        body,
        grid=(n // n_blk,),
        in_specs=[
            pl.BlockSpec((n_blk, d), lambda i: (i, 0)),
            pl.BlockSpec((d, d), lambda i: (0, 0)),
            pl.BlockSpec((t, d), lambda i: (0, 0)),
        ],
        out_specs=pl.BlockSpec((n_blk, t, d), lambda i: (i, 0, 0)),
        out_shape=jax.ShapeDtypeStruct((n, t, d), jnp.float32),
    )(x, W, b_type)


def _final_call(partials, x, W_self, *, n_blk):
    n, d = x.shape

    def body(p_ref, x_ref, w_ref, out_ref):
        s = lax.dot(
            x_ref[...],
            w_ref[...],
            precision=lax.Precision.HIGHEST,
            preferred_element_type=jnp.float32,
        )
        out_ref[...] = jnp.maximum(p_ref[0] + p_ref[1] + s, 0.0)

    return pl.pallas_call(
        body,
        grid=(n // n_blk,),
        in_specs=[
            pl.BlockSpec((2, n_blk, d), lambda i: (0, i, 0)),
            pl.BlockSpec((n_blk, d), lambda i: (i, 0)),
            pl.BlockSpec((d, d), lambda i: (0, 0)),
        ],
        out_specs=pl.BlockSpec((n_blk, d), lambda i: (i, 0)),
        out_shape=jax.ShapeDtypeStruct((n, d), jnp.float32),
    )(partials, x, W_self)


def _sc_aggregate(htab2, gidx3, dst3, *, n, d, n_ch, ch):
    npt = (n // _NS) // 8 * 8
    rem = n - _NS * npt
    zch = 104
    assert npt % zch == 0 and rem % 8 == 0 and rem <= ch
    mesh = plsc.VectorSubcoreMesh(
        core_axis_name="c", subcore_axis_name="s", num_cores=_NC, num_subcores=_NS
    )

    @functools.partial(
        pl.kernel,
        mesh=mesh,
        out_type=jax.ShapeDtypeStruct((_NC, n, d), jnp.float32),
        scratch_types=[
            pltpu.VMEM((n_ch, ch), jnp.int32),
            pltpu.VMEM((n_ch, ch), jnp.int32),
            pltpu.VMEM((ch, d), jnp.float32),
            pltpu.VMEM_SHARED((n, d), jnp.float32),
            pltpu.SemaphoreType.DMA,
        ],
    )
    def run(tab_hbm, gidx_hbm, dst_hbm, out_hbm, gidx_v, dst_v, rows_v, acc_sh, sem):
        cid = lax.axis_index("c")
        sid = lax.axis_index("s")
        wid = cid * _NS + sid

        pltpu.sync_copy(gidx_hbm.at[wid], gidx_v)
        pltpu.sync_copy(dst_hbm.at[wid], dst_v)

        def zero_row(i, carry):
            for c in range(d // 16):
                rows_v[i, pl.ds(c * 16, 16)] = jnp.zeros((16,), jnp.float32)
            return carry

        lax.fori_loop(0, ch, zero_row, 0)
        for k in range(npt // zch):
            pltpu.sync_copy(
                rows_v.at[pl.ds(0, zch)],
                acc_sh.at[pl.ds(sid * npt + k * zch, zch)],
            )

        @pl.when(sid == _NS - 1)
        def _zero_tail():
            pltpu.sync_copy(
                rows_v.at[pl.ds(0, rem)], acc_sh.at[pl.ds(_NS * npt, rem)]
            )

        plsc.subcore_barrier()

        def chunk(j, carry):
            pltpu.async_copy(tab_hbm.at[gidx_v.at[j]], rows_v, sem).wait()
            pltpu.sync_copy(rows_v, acc_sh.at[dst_v.at[j]], add=True)
            return carry

        lax.fori_loop(0, n_ch, chunk, 0)
        plsc.subcore_barrier()

        pltpu.sync_copy(
            acc_sh.at[pl.ds(sid * npt, npt)],
            out_hbm.at[cid, pl.ds(sid * npt, npt)],
        )

        @pl.when(sid == _NS - 1)
        def _write_tail():
            pltpu.sync_copy(
                acc_sh.at[pl.ds(_NS * npt, rem)],
                out_hbm.at[cid, pl.ds(_NS * npt, rem)],
            )

    return run(htab2, gidx3, dst3)


def kernel(x, edge_index, edge_type, W, W_self, b_type):
    n, d = x.shape
    e = edge_index.shape[1]
    t = b_type.shape[0]
    assert e % _NW == 0
    epw = e // _NW
    ch = 125
    n_ch = epw // ch
    assert n_ch * ch == epw

    src = edge_index[0]
    dst = edge_index[1]
    gidx = src * t + edge_type

    htab = _htab_call(x, W, b_type, n_blk=1000)
    partials = _sc_aggregate(
        htab.reshape(n * t, d),
        gidx.reshape(_NW, n_ch, ch),
        dst.reshape(_NW, n_ch, ch),
        n=n, d=d, n_ch=n_ch, ch=ch,
    )
    return _final_call(partials, x, W_self, n_blk=1000)


# speedup vs baseline: 8.6564x; 1.0038x over previous
import functools

import jax
import jax.numpy as jnp
from jax import lax
from jax.experimental import pallas as pl
from jax.experimental.pallas import tpu as pltpu
from jax.experimental.pallas import tpu_sc as plsc

_NC = 2
_NS = 16
_NW = _NC * _NS
_CH = 128
_DSTBITS = 14


def _htab_call(x, W, b_type, *, n_blk):
    n, d = x.shape
    t = b_type.shape[0]

    def body(x_ref, w_ref, b_ref, out_ref):
        h = lax.dot(
            x_ref[...],
            w_ref[...],
            precision=lax.Precision.HIGHEST,
            preferred_element_type=jnp.float32,
        )
        out_ref[...] = (h[:, None, :] + b_ref[...][None, :, :]).reshape(
            n_blk * t, d
        )

    return pl.pallas_call(
        body,
        grid=(n // n_blk,),
        in_specs=[
            pl.BlockSpec((n_blk, d), lambda i: (i, 0)),
            pl.BlockSpec((d, d), lambda i: (0, 0)),
            pl.BlockSpec((t, d), lambda i: (0, 0)),
        ],
        out_specs=pl.BlockSpec((n_blk * t, d), lambda i: (i, 0)),
        out_shape=jax.ShapeDtypeStruct((n * t, d), jnp.float32),
    )(x, W, b_type)


def _final_call(partials, x, W_self, *, n_blk):
    n, d = x.shape

    def body(p_ref, x_ref, w_ref, out_ref):
        s = lax.dot(
            x_ref[...],
            w_ref[...],
            precision=lax.Precision.HIGHEST,
            preferred_element_type=jnp.float32,
        )
        out_ref[...] = jnp.maximum(p_ref[0] + p_ref[1] + s, 0.0)

    return pl.pallas_call(
        body,
        grid=(n // n_blk,),
        in_specs=[
            pl.BlockSpec((2, n_blk, d), lambda i: (0, i, 0)),
            pl.BlockSpec((n_blk, d), lambda i: (i, 0)),
            pl.BlockSpec((d, d), lambda i: (0, 0)),
        ],
        out_specs=pl.BlockSpec((n_blk, d), lambda i: (i, 0)),
        out_shape=jax.ShapeDtypeStruct((n, d), jnp.float32),
    )(partials, x, W_self)


def _sc_aggregate(htab2, packed3, *, n, n_acc, d, n_ch):
    ch = _CH
    npt = (n // _NS) // 8 * 8
    wrem = n - _NS * npt
    zrem = n_acc - _NS * npt
    zch = 104
    assert npt % zch == 0 and wrem % 8 == 0 and zrem % 8 == 0
    assert max(wrem, zrem) <= ch and zch <= ch
    mesh = plsc.VectorSubcoreMesh(
        core_axis_name="c", subcore_axis_name="s", num_cores=_NC, num_subcores=_NS
    )

    @functools.partial(
        pl.kernel,
        mesh=mesh,
        out_type=jax.ShapeDtypeStruct((_NC, n, d), jnp.float32),
        scratch_types=[
            pltpu.VMEM((n_ch, ch), jnp.int32),
            pltpu.VMEM((2, ch), jnp.int32),
            pltpu.VMEM((2, ch), jnp.int32),
            pltpu.VMEM((2, ch, d), jnp.float32),
            pltpu.VMEM_SHARED((n_acc, d), jnp.float32),
            pltpu.SemaphoreType.DMA((2,)),
        ],
    )
    def run(tab_hbm, pk_hbm, out_hbm, pk_v, gi_v, di_v, rows2_v, acc_sh, sems):
        cid = lax.axis_index("c")
        sid = lax.axis_index("s")
        wid = cid * _NS + sid

        pltpu.sync_copy(pk_hbm.at[wid], pk_v)

        def zero_row(i, carry):
            for c in range(d // 16):
                rows2_v[0, i, pl.ds(c * 16, 16)] = jnp.zeros((16,), jnp.float32)
            return carry

        lax.fori_loop(0, zch, zero_row, 0)
        for k in range(npt // zch):
            pltpu.sync_copy(
                rows2_v.at[0, pl.ds(0, zch)],
                acc_sh.at[pl.ds(sid * npt + k * zch, zch)],
            )

        @pl.when(sid == _NS - 1)
        def _zero_tail():
            pltpu.sync_copy(
                rows2_v.at[0, pl.ds(0, zrem)], acc_sh.at[pl.ds(_NS * npt, zrem)]
            )

        plsc.subcore_barrier()

        def unpack(j, buf):
            for c in range(ch // 16):
                w = pk_v[j, pl.ds(c * 16, 16)]
                gi_v[buf, pl.ds(c * 16, 16)] = lax.shift_right_logical(
                    w, _DSTBITS
                )
                di_v[buf, pl.ds(c * 16, 16)] = lax.bitwise_and(
                    w, (1 << _DSTBITS) - 1
                )

        def start_gather(buf):
            return pltpu.async_copy(
                tab_hbm.at[gi_v.at[buf]], rows2_v.at[buf], sems.at[buf]
            )

        def wait_gather(buf):
            pltpu.make_async_copy(
                tab_hbm.at[gi_v.at[buf]], rows2_v.at[buf], sems.at[buf]
            ).wait()

        def scatter(buf):
            pltpu.sync_copy(rows2_v.at[buf], acc_sh.at[di_v.at[buf]], add=True)

        unpack(0, 0)
        start_gather(0)

        def pair(k, carry):
            j = 2 * k

            @pl.when(j + 1 < n_ch)
            def _p1():
                unpack(j + 1, 1)
                start_gather(1)

            wait_gather(0)
            scatter(0)

            @pl.when(j + 2 < n_ch)
            def _p2():
                unpack(j + 2, 0)
                start_gather(0)

            @pl.when(j + 1 < n_ch)
            def _c1():
                wait_gather(1)
                scatter(1)

            return carry

        lax.fori_loop(0, (n_ch + 1) // 2, pair, 0)
        plsc.subcore_barrier()

        pltpu.sync_copy(
            acc_sh.at[pl.ds(sid * npt, npt)],
            out_hbm.at[cid, pl.ds(sid * npt, npt)],
        )

        @pl.when(sid == _NS - 1)
        def _write_tail():
            pltpu.sync_copy(
                acc_sh.at[pl.ds(_NS * npt, wrem)],
                out_hbm.at[cid, pl.ds(_NS * npt, wrem)],
            )

    return run(htab2, packed3)


def kernel(x, edge_index, edge_type, W, W_self, b_type):
    n, d = x.shape
    e = edge_index.shape[1]
    t = b_type.shape[0]
    assert e % _NW == 0 and n < (1 << _DSTBITS)
    epw = e // _NW
    n_ch = -(-epw // _CH)
    pad = n_ch * _CH - epw
    n_acc = -(-(n + 1) // 8) * 8

    src = edge_index[0]
    dst = edge_index[1]
    gidx = src * t + edge_type
    packed = (gidx << _DSTBITS) | dst
    packed = packed.reshape(_NW, epw)
    if pad:
        filler = jnp.full((_NW, pad), n, dtype=jnp.int32)
        packed = jnp.concatenate([packed, filler], axis=1)
    packed3 = packed.reshape(_NW, n_ch, _CH)

    htab = _htab_call(x, W, b_type, n_blk=1000)
    partials = _sc_aggregate(htab, packed3, n=n, n_acc=n_acc, d=d, n_ch=n_ch)
    return _final_call(partials, x, W_self, n_blk=1000)
